# baseline (speedup 1.0000x reference)
import functools

import jax
import jax.numpy as jnp
import numpy as np
from jax.experimental import pallas as pl
from jax.experimental.pallas import tpu as pltpu


def _fused_kernel(x_ref, d_ref, w_ref, b_ref, o_ref, s_ref, *, width):
    c = x_ref.shape[1]
    hw = x_ref.shape[2]
    h = hw // width
    w2 = 2 * width
    hw2 = 2 * hw

    col = jax.lax.broadcasted_iota(jnp.int32, (1, hw2), 1) % w2
    left_ok = (col >= 1).astype(jnp.bfloat16)
    right_ok = (col <= w2 - 2).astype(jnp.bfloat16)
    zeros_edge = jnp.zeros((c, w2), jnp.bfloat16)

    bias = b_ref[...].astype(jnp.float32)
    for im in (0, 1):
        x = x_ref[im].astype(jnp.bfloat16)
        xc = jnp.dot(x, d_ref[...],
                     preferred_element_type=jnp.float32).astype(jnp.bfloat16)
        y = {0: xc}
        y[-1] = jnp.concatenate(
            [jnp.zeros((c, 1), xc.dtype), xc[:, :hw2 - 1]], axis=1) * left_ok
        y[1] = jnp.concatenate(
            [xc[:, 1:], jnp.zeros((c, 1), xc.dtype)], axis=1) * right_ok
        col0 = hw2 * im
        for d in (-1, 0, 1):
            for r in (-1, 0, 1):
                row0 = c * ((r + 1) * 3 + (d + 1))
                if r == 0:
                    s_ref[row0:row0 + c, col0:col0 + hw2] = y[d]
                elif r == 1:
                    s_ref[row0:row0 + c, col0:col0 + hw2 - w2] = y[d][:, w2:]
                    s_ref[row0:row0 + c, col0 + hw2 - w2:col0 + hw2] = \
                        zeros_edge
                else:
                    s_ref[row0:row0 + c, col0:col0 + w2] = zeros_edge
                    s_ref[row0:row0 + c, col0 + w2:col0 + hw2] = \
                        y[d][:, :hw2 - w2]

        t = []
        for p in (0, 1):
            a = jnp.dot(w_ref[p],
                        s_ref[3 * c * p:3 * c * p + 6 * c, col0:col0 + hw2],
                        preferred_element_type=jnp.float32)
            t.append((a + bias).astype(o_ref.dtype))

        for a_ in range(h):
            o_ref[im, :, 2 * w2 * a_:2 * w2 * a_ + w2] = \
                t[0][:, w2 * a_:w2 * (a_ + 1)]
            o_ref[im, :, 2 * w2 * a_ + w2:2 * w2 * (a_ + 1)] = \
                t[1][:, w2 * a_:w2 * (a_ + 1)]


def kernel(x, conv_weight, conv_bias):
    n, c, h, w = x.shape
    hw = h * w

    k_i = np.arange(hw)[:, None]
    m_i = np.arange(2 * hw)[None, :]
    src = (m_i // (2 * w)) * w + (m_i % (2 * w)) // 2
    d = jnp.asarray(k_i == src, dtype=jnp.bfloat16)

    A = jnp.array([[[1., 0., 0.], [0., 1., 1.]],
                   [[1., 1., 0.], [0., 0., 1.]]], jnp.float32)
    w2c = jnp.einsum('puy,oiyx->puxoi', A, conv_weight)
    wp = jnp.concatenate([w2c[:, u, kx] for u in (0, 1) for kx in range(3)],
                         axis=2).astype(jnp.bfloat16)
    b2 = conv_bias.reshape(c, 1)
    x2 = x.reshape(n // 2, 2, c, hw)

    out = pl.pallas_call(
        functools.partial(_fused_kernel, width=w),
        out_shape=jax.ShapeDtypeStruct((n // 2, 2, c, 4 * hw), jnp.float32),
        grid=(n // 2,),
        in_specs=[
            pl.BlockSpec((None, 2, c, hw), lambda i: (i, 0, 0, 0)),
            pl.BlockSpec((hw, 2 * hw), lambda i: (0, 0)),
            pl.BlockSpec((2, c, 6 * c), lambda i: (0, 0, 0)),
            pl.BlockSpec((c, 1), lambda i: (0, 0)),
        ],
        out_specs=pl.BlockSpec((None, 2, c, 4 * hw), lambda i: (i, 0, 0, 0)),
        scratch_shapes=[pltpu.VMEM((9 * c, 4 * hw), jnp.bfloat16)],
        compiler_params=pltpu.CompilerParams(
            dimension_semantics=("parallel",),
            vmem_limit_bytes=64 * 1024 * 1024,
        ),
    )(x2, d, wp, b2)
    return out.reshape(n, c, 2 * h, 2 * w)


# speedup vs baseline: 1.0612x; 1.0612x over previous
import functools

import jax
import jax.numpy as jnp
import numpy as np
from jax.experimental import pallas as pl
from jax.experimental.pallas import tpu as pltpu


def _fused_kernel(x_ref, d_ref, w_ref, b_ref, o_ref, s_ref, *, width):
    c = x_ref.shape[1]
    hw = x_ref.shape[2]
    h = hw // width
    w2 = 2 * width
    hw2 = 2 * hw

    col = jax.lax.broadcasted_iota(jnp.int32, (1, hw2), 1) % w2
    left_ok = (col >= 1).astype(jnp.bfloat16)
    right_ok = (col <= w2 - 2).astype(jnp.bfloat16)
    zeros_edge = jnp.zeros((c, w2), jnp.bfloat16)

    for im in (0, 1, 2, 3):
        x = x_ref[im].astype(jnp.bfloat16)
        xc = jnp.dot(x, d_ref[...],
                     preferred_element_type=jnp.float32).astype(jnp.bfloat16)
        y = {0: xc}
        y[-1] = jnp.concatenate(
            [jnp.zeros((c, 1), xc.dtype), xc[:, :hw2 - 1]], axis=1) * left_ok
        y[1] = jnp.concatenate(
            [xc[:, 1:], jnp.zeros((c, 1), xc.dtype)], axis=1) * right_ok
        col0 = hw2 * im
        for d in (-1, 0, 1):
            for r in (-1, 0, 1):
                row0 = c * ((r + 1) * 3 + (d + 1))
                if r == 0:
                    s_ref[row0:row0 + c, col0:col0 + hw2] = y[d]
                elif r == 1:
                    s_ref[row0:row0 + c, col0:col0 + hw2 - w2] = y[d][:, w2:]
                    s_ref[row0:row0 + c, col0 + hw2 - w2:col0 + hw2] = \
                        zeros_edge
                else:
                    s_ref[row0:row0 + c, col0:col0 + w2] = zeros_edge
                    s_ref[row0:row0 + c, col0 + w2:col0 + hw2] = \
                        y[d][:, :hw2 - w2]

    bias = b_ref[...].astype(jnp.float32)
    t = []
    for p in (0, 1):
        a = jnp.dot(w_ref[p], s_ref[3 * c * p:3 * c * p + 6 * c, :],
                    preferred_element_type=jnp.float32)
        t.append((a + bias).astype(o_ref.dtype))

    for im in (0, 1, 2, 3):
        col0 = hw2 * im
        for a_ in range(h):
            o_ref[im, :, 2 * w2 * a_:2 * w2 * a_ + w2] = \
                t[0][:, col0 + w2 * a_:col0 + w2 * (a_ + 1)]
            o_ref[im, :, 2 * w2 * a_ + w2:2 * w2 * (a_ + 1)] = \
                t[1][:, col0 + w2 * a_:col0 + w2 * (a_ + 1)]


def kernel(x, conv_weight, conv_bias):
    n, c, h, w = x.shape
    hw = h * w

    k_i = np.arange(hw)[:, None]
    m_i = np.arange(2 * hw)[None, :]
    src = (m_i // (2 * w)) * w + (m_i % (2 * w)) // 2
    d = jnp.asarray(k_i == src, dtype=jnp.bfloat16)

    A = jnp.array([[[1., 0., 0.], [0., 1., 1.]],
                   [[1., 1., 0.], [0., 0., 1.]]], jnp.float32)
    w2c = jnp.einsum('puy,oiyx->puxoi', A, conv_weight)
    wp = jnp.concatenate([w2c[:, u, kx] for u in (0, 1) for kx in range(3)],
                         axis=2).astype(jnp.bfloat16)
    b2 = conv_bias.reshape(c, 1)
    x2 = x.reshape(n // 4, 4, c, hw)

    out = pl.pallas_call(
        functools.partial(_fused_kernel, width=w),
        out_shape=jax.ShapeDtypeStruct((n // 4, 4, c, 4 * hw), jnp.float32),
        grid=(n // 4,),
        in_specs=[
            pl.BlockSpec((None, 4, c, hw), lambda i: (i, 0, 0, 0)),
            pl.BlockSpec((hw, 2 * hw), lambda i: (0, 0)),
            pl.BlockSpec((2, c, 6 * c), lambda i: (0, 0, 0)),
            pl.BlockSpec((c, 1), lambda i: (0, 0)),
        ],
        out_specs=pl.BlockSpec((None, 4, c, 4 * hw), lambda i: (i, 0, 0, 0)),
        scratch_shapes=[pltpu.VMEM((9 * c, 8 * hw), jnp.bfloat16)],
        compiler_params=pltpu.CompilerParams(
            dimension_semantics=("parallel",),
            vmem_limit_bytes=64 * 1024 * 1024,
        ),
    )(x2, d, wp, b2)
    return out.reshape(n, c, 2 * h, 2 * w)
